# Initial kernel scaffold; baseline (speedup 1.0000x reference)
import jax
import jax.numpy as jnp
from jax import lax
from jax.experimental import pallas as pl
from jax.experimental.pallas import tpu as pltpu
from jax.experimental.pallas import tpu_sc as plsc

N_ENT_ = 100000
N_REL_ = 1000
D_ = 128
H_ = 128
N_SEL_ = 8192
NE_ = 524288

NW = 32
BW = N_SEL_ // NW
CH = 128


def _sc_body(ne_hbm, nr_hbm, off_hbm, sel_hbm, et_hbm, rt_hbm,
             out_e, out_r, out_rows,
             off_v, idx_e, idx_r, rows_e, rows_r, acc_e, acc_r,
             sem_a, sem_b):
    cid = lax.axis_index("c")
    sid = lax.axis_index("s")
    wid = sid * 2 + cid
    bag0 = wid * BW

    pltpu.sync_copy(off_hbm.at[pl.ds(bag0, BW + 8)], off_v)
    e0 = off_v[0]
    e1 = off_v[BW]

    def zero_body(i, carry):
        zv = jnp.zeros((16,), jnp.float32)
        for cc in range(8):
            acc_e[i, pl.ds(cc * 16, 16)] = zv
            acc_r[i, pl.ds(cc * 16, 16)] = zv
        return carry
    lax.fori_loop(0, BW, zero_body, 0)

    for half in range(BW // CH):
        base = bag0 + half * CH
        pltpu.sync_copy(sel_hbm.at[pl.ds(base, CH)], idx_e)
        pltpu.async_copy(et_hbm.at[idx_e], rows_e, sem_a).wait()
        pltpu.sync_copy(rows_e, out_rows.at[pl.ds(base, CH)])

    start = (e0 // CH) * CH
    nchunks = (e1 - start + CH - 1) // CH

    def chunk_body(ci, bp):
        s = start + ci * CH
        pltpu.sync_copy(ne_hbm.at[pl.ds(s, CH)], idx_e)
        pltpu.sync_copy(nr_hbm.at[pl.ds(s, CH)], idx_r)
        cp_a = pltpu.async_copy(et_hbm.at[idx_e], rows_e, sem_a)
        cp_b = pltpu.async_copy(rt_hbm.at[idx_r], rows_r, sem_b)
        cp_a.wait()
        cp_b.wait()

        def edge_body(j, bp):
            e = s + j
            valid = jnp.logical_and(e >= e0, e < e1)

            def adv(b):
                return lax.while_loop(lambda bb: off_v[bb + 1] <= e,
                                      lambda bb: bb + 1, b)
            bp = lax.cond(valid, adv, lambda b: b, bp)

            @pl.when(valid)
            def _():
                for cc in range(8):
                    sl = pl.ds(cc * 16, 16)
                    plsc.addupdate(acc_e.at[bp, sl], rows_e[j, sl])
                    plsc.addupdate(acc_r.at[bp, sl], rows_r[j, sl])
            return bp

        return lax.fori_loop(0, CH, edge_body, bp)

    lax.fori_loop(0, nchunks, chunk_body, 0)

    pltpu.sync_copy(acc_e, out_e.at[pl.ds(bag0, BW)])
    pltpu.sync_copy(acc_r, out_r.at[pl.ds(bag0, BW)])


def _sc_bags(neighbor_entities, neighbor_relations, offsets_ext, entities,
             entity_table, relation_table):
    f32 = jnp.float32
    mesh = plsc.VectorSubcoreMesh(core_axis_name="c", subcore_axis_name="s")
    out_type = (jax.ShapeDtypeStruct((N_SEL_, D_), f32),
                jax.ShapeDtypeStruct((N_SEL_, D_), f32),
                jax.ShapeDtypeStruct((N_SEL_, D_), f32))
    scratch = [
        pltpu.VMEM((BW + 8,), jnp.int32),
        pltpu.VMEM((CH,), jnp.int32),
        pltpu.VMEM((CH,), jnp.int32),
        pltpu.VMEM((CH, D_), f32),
        pltpu.VMEM((CH, D_), f32),
        pltpu.VMEM((BW, D_), f32),
        pltpu.VMEM((BW, D_), f32),
        pltpu.SemaphoreType.DMA,
        pltpu.SemaphoreType.DMA,
    ]
    return pl.kernel(_sc_body, out_type=out_type, mesh=mesh,
                     scratch_types=scratch)(
        neighbor_entities, neighbor_relations, offsets_ext, entities,
        entity_table, relation_table)


def _tc_node_body(rows_ref, se_ref, sr_ref, off0_ref, off1_ref,
                  wtT_ref, wnaT_ref, wnbT_ref, bsum_ref, out_ref):
    cnt = (off1_ref[...] - off0_ref[...]).astype(jnp.float32)
    inv = 1.0 / jnp.maximum(cnt, 1.0)
    mean_e = se_ref[...] * inv
    mean_r = sr_ref[...] * inv
    acc = jnp.dot(rows_ref[...], wtT_ref[...],
                  preferred_element_type=jnp.float32)
    acc += jnp.dot(mean_e, wnaT_ref[...], preferred_element_type=jnp.float32)
    acc += jnp.dot(mean_r, wnbT_ref[...], preferred_element_type=jnp.float32)
    out_ref[...] = jnp.maximum(acc + bsum_ref[...], 0.0)


def _tc_pair_body(node_ref, wrT_ref, wpT_ref, br_ref, bpp_ref, out_ref):
    pair = jnp.dot(node_ref[...], wrT_ref[...],
                   preferred_element_type=jnp.float32) + br_ref[...]
    pair = jnp.maximum(pair, 0.0)
    out_ref[...] = jnp.dot(pair, wpT_ref[...],
                           preferred_element_type=jnp.float32) + bpp_ref[...]


def kernel(entities, neighbor_entities, neighbor_relations, offsets,
           entity_table, relation_table, Wn, bn, Wt, bt, Wr, br, Wp, bp):
    f32 = jnp.float32
    offsets_ext = jnp.concatenate(
        [offsets, jnp.full((8,), NE_, jnp.int32)])

    sum_e, sum_r, rows = _sc_bags(neighbor_entities, neighbor_relations,
                                  offsets_ext, entities,
                                  entity_table, relation_table)

    off0 = offsets_ext[:N_SEL_, None]
    off1 = offsets_ext[1:N_SEL_ + 1, None]
    wtT = Wt.T
    wnaT = Wn[:, :D_].T
    wnbT = Wn[:, D_:].T
    bsum = (bt + bn)[None, :]

    node = pl.pallas_call(
        _tc_node_body,
        out_shape=jax.ShapeDtypeStruct((N_SEL_, H_), f32),
    )(rows, sum_e, sum_r, off0, off1, wtT, wnaT, wnbT, bsum)

    node2 = node.reshape(N_SEL_ // 2, 2 * H_)
    wrT = Wr.T
    wpT = jnp.pad(Wp, ((0, 1), (0, 0))).T
    bpp = jnp.pad(bp, (0, 1))[None, :]

    scores = pl.pallas_call(
        _tc_pair_body,
        out_shape=jax.ShapeDtypeStruct((N_SEL_ // 2, N_REL_), f32),
    )(node2, wrT, wpT, br[None, :], bpp)

    return scores[:, :N_REL_ - 1]


# speedup vs baseline: 109.8831x; 109.8831x over previous
import jax
import jax.numpy as jnp
from jax import lax
from jax.experimental import pallas as pl
from jax.experimental.pallas import tpu as pltpu
from jax.experimental.pallas import tpu_sc as plsc

N_ENT_ = 100000
N_REL_ = 1000
D_ = 128
H_ = 128
N_SEL_ = 8192
NE_ = 524288

NW = 32
EW = NE_ // NW
CH = 128
NCH = EW // CH
SEL_W = N_SEL_ // NW
ACC_SLICE = N_SEL_ // 16


def _sc_body(ne_hbm, nr_hbm, off_hbm, sel_hbm, et_hbm, rt_hbm,
             pe_hbm, pr_hbm, rows_out_hbm,
             off_v, seg_v, idx_v, rows_v, zeros_v, acc_sh, tab_sh, sem_g):
    cid = lax.axis_index("c")
    sid = lax.axis_index("s")
    wid = cid * 16 + sid
    e_lo = wid * EW

    pltpu.sync_copy(off_hbm, off_v)

    zf = jnp.zeros((16,), jnp.float32)

    @pl.loop(0, CH)
    def _(i):
        for c8 in range(8):
            zeros_v[i, pl.ds(c8 * 16, 16)] = zf

    @pl.when(sid < 7)
    def _():
        pltpu.sync_copy(rt_hbm.at[pl.ds(sid * 128, 128)],
                        tab_sh.at[pl.ds(sid * 128, 128)])

    @pl.when(sid == 7)
    def _():
        pltpu.sync_copy(rt_hbm.at[pl.ds(7 * 128, N_REL_ - 7 * 128)],
                        tab_sh.at[pl.ds(7 * 128, N_REL_ - 7 * 128)])

    for q in range(ACC_SLICE // CH):
        pltpu.sync_copy(zeros_v,
                        acc_sh.at[pl.ds(sid * ACC_SLICE + q * CH, CH)])

    zi = jnp.zeros((16,), jnp.int32)

    @pl.loop(0, NCH)
    def _(i):
        for c8 in range(8):
            seg_v[i, pl.ds(c8 * 16, 16)] = zi

    ones = jnp.full((16,), 1, jnp.int32)

    def mk_body(k, bcnt):
        offv = off_v[pl.ds(k * 16, 16)]
        inside = jnp.logical_and(offv > e_lo, offv < e_lo + EW)
        p = offv - e_lo
        prow = p // CH
        pcol = p - prow * CH
        plsc.addupdate_scatter(seg_v, [prow, pcol], ones, mask=inside)
        return bcnt + jnp.where(offv <= e_lo, 1, 0)

    bcnt = pl.loop(0, N_SEL_ // 16,
                   init_carry=jnp.zeros((16,), jnp.int32))(mk_body)
    base = jnp.sum(bcnt) - 1

    def cs_body(i, carry):
        for c8 in range(8):
            v = seg_v[i, pl.ds(c8 * 16, 16)]
            c = plsc.cumsum(v) + carry
            seg_v[i, pl.ds(c8 * 16, 16)] = c + base
            carry = c[15]
        return carry

    pl.loop(0, NCH, init_carry=jnp.int32(0))(cs_body)

    for h in range(SEL_W // CH):
        b = wid * SEL_W + h * CH
        pltpu.sync_copy(sel_hbm.at[pl.ds(b, CH)], idx_v)
        pltpu.async_copy(et_hbm.at[idx_v], rows_v, sem_g).wait()
        pltpu.sync_copy(rows_v, rows_out_hbm.at[pl.ds(b, CH)])

    plsc.subcore_barrier()

    @pl.loop(0, NCH)
    def _(ci):
        s = e_lo + ci * CH
        pltpu.sync_copy(ne_hbm.at[pl.ds(s, CH)], idx_v)
        pltpu.async_copy(et_hbm.at[idx_v], rows_v, sem_g).wait()
        pltpu.sync_copy(rows_v, acc_sh.at[seg_v.at[ci]], add=True)

    plsc.subcore_barrier()
    o = cid * N_SEL_ + sid * ACC_SLICE
    pltpu.sync_copy(acc_sh.at[pl.ds(sid * ACC_SLICE, ACC_SLICE)],
                    pe_hbm.at[pl.ds(o, ACC_SLICE)])
    for q in range(ACC_SLICE // CH):
        pltpu.sync_copy(zeros_v,
                        acc_sh.at[pl.ds(sid * ACC_SLICE + q * CH, CH)])
    plsc.subcore_barrier()

    @pl.loop(0, NCH)
    def _(ci):
        s = e_lo + ci * CH
        pltpu.sync_copy(nr_hbm.at[pl.ds(s, CH)], idx_v)
        pltpu.async_copy(tab_sh.at[idx_v], rows_v, sem_g).wait()
        pltpu.sync_copy(rows_v, acc_sh.at[seg_v.at[ci]], add=True)

    plsc.subcore_barrier()
    pltpu.sync_copy(acc_sh.at[pl.ds(sid * ACC_SLICE, ACC_SLICE)],
                    pr_hbm.at[pl.ds(o, ACC_SLICE)])


def _sc_bags(neighbor_entities, neighbor_relations, offsets, entities,
             entity_table, relation_table):
    f32 = jnp.float32
    mesh = plsc.VectorSubcoreMesh(core_axis_name="c", subcore_axis_name="s")
    out_type = (jax.ShapeDtypeStruct((2 * N_SEL_, D_), f32),
                jax.ShapeDtypeStruct((2 * N_SEL_, D_), f32),
                jax.ShapeDtypeStruct((N_SEL_, D_), f32))
    scratch = [
        pltpu.VMEM((N_SEL_,), jnp.int32),
        pltpu.VMEM((NCH, CH), jnp.int32),
        pltpu.VMEM((CH,), jnp.int32),
        pltpu.VMEM((CH, D_), f32),
        pltpu.VMEM((CH, D_), f32),
        pltpu.VMEM_SHARED((N_SEL_, D_), f32),
        pltpu.VMEM_SHARED((N_REL_, D_), f32),
        pltpu.SemaphoreType.DMA,
    ]
    return pl.kernel(
        _sc_body, out_type=out_type, mesh=mesh, scratch_types=scratch,
        compiler_params=pltpu.CompilerParams(needs_layout_passes=False))(
        neighbor_entities, neighbor_relations, offsets, entities,
        entity_table, relation_table)


def _tc_node_body(r_ref, pe0_ref, pe1_ref, pr0_ref, pr1_ref,
                  off0_ref, off1_ref, wtT_ref, wnaT_ref, wnbT_ref,
                  bsum_ref, out_ref):
    cnt = (off1_ref[...] - off0_ref[...]).astype(jnp.float32)
    inv = 1.0 / jnp.maximum(cnt, 1.0)
    mean_e = (pe0_ref[...] + pe1_ref[...]) * inv
    mean_r = (pr0_ref[...] + pr1_ref[...]) * inv
    acc = jnp.dot(r_ref[...], wtT_ref[...],
                  preferred_element_type=jnp.float32)
    acc += jnp.dot(mean_e, wnaT_ref[...], preferred_element_type=jnp.float32)
    acc += jnp.dot(mean_r, wnbT_ref[...], preferred_element_type=jnp.float32)
    out_ref[...] = jnp.maximum(acc + bsum_ref[...], 0.0)


def _tc_pair_body(node_ref, wrT_ref, wpT_ref, br_ref, bpp_ref, out_ref):
    pair = jnp.dot(node_ref[...], wrT_ref[...],
                   preferred_element_type=jnp.float32) + br_ref[...]
    pair = jnp.maximum(pair, 0.0)
    out_ref[...] = jnp.dot(pair, wpT_ref[...],
                           preferred_element_type=jnp.float32) + bpp_ref[...]


def kernel(entities, neighbor_entities, neighbor_relations, offsets,
           entity_table, relation_table, Wn, bn, Wt, bt, Wr, br, Wp, bp):
    f32 = jnp.float32

    part_e, part_r, rows = _sc_bags(neighbor_entities, neighbor_relations,
                                    offsets, entities,
                                    entity_table, relation_table)

    offsets_ext = jnp.concatenate([offsets, jnp.full((8,), NE_, jnp.int32)])
    off0 = offsets_ext[:N_SEL_, None]
    off1 = offsets_ext[1:N_SEL_ + 1, None]
    wtT = Wt.T
    wnaT = Wn[:, :D_].T
    wnbT = Wn[:, D_:].T
    bsum = (bt + bn)[None, :]

    node = pl.pallas_call(
        _tc_node_body,
        out_shape=jax.ShapeDtypeStruct((N_SEL_, H_), f32),
    )(rows, part_e[:N_SEL_], part_e[N_SEL_:], part_r[:N_SEL_],
      part_r[N_SEL_:], off0, off1, wtT, wnaT, wnbT, bsum)

    node2 = node.reshape(N_SEL_ // 2, 2 * H_)
    wrT = Wr.T
    wpT = jnp.pad(Wp, ((0, 1), (0, 0))).T
    bpp = jnp.pad(bp, (0, 1))[None, :]

    scores = pl.pallas_call(
        _tc_pair_body,
        out_shape=jax.ShapeDtypeStruct((N_SEL_ // 2, N_REL_), f32),
    )(node2, wrT, wpT, br[None, :], bpp)

    return scores[:, :N_REL_ - 1]


# speedup vs baseline: 159.1321x; 1.4482x over previous
import jax
import jax.numpy as jnp
from jax import lax
from jax.experimental import pallas as pl
from jax.experimental.pallas import tpu as pltpu
from jax.experimental.pallas import tpu_sc as plsc

N_ENT_ = 100000
N_REL_ = 1000
D_ = 128
H_ = 128
N_SEL_ = 8192
NE_ = 524288

NW = 32
EW = NE_ // NW
CH = 128
NCH = EW // CH
SEL_W = N_SEL_ // NW
ACC_SLICE = N_SEL_ // 16
NBUF = 2


def _sc_body(ne_hbm, nr_hbm, off_hbm, sel_hbm, et_hbm, rt_hbm,
             pe_hbm, pr_hbm, rows_out_hbm,
             off_win, seg_v, ei0, ei1, idx1_v, rows0, rows1,
             acc_sh, tab_sh,
             sg0, sg1, ss0, ss1, si0, si1):
    cid = lax.axis_index("c")
    sid = lax.axis_index("s")
    wid = cid * 16 + sid
    e_lo = wid * EW
    rows_b = (rows0, rows1)
    ei_b = (ei0, ei1)
    sem_g = (sg0, sg1)
    sem_s = (ss0, ss1)
    sem_i = (si0, si1)

    def drain(sem, ref):
        pltpu.make_async_copy(et_hbm.at[pl.ds(0, CH)], ref, sem).wait()

    @pl.when(sid < 7)
    def _():
        pltpu.sync_copy(rt_hbm.at[pl.ds(sid * 128, 128)],
                        tab_sh.at[pl.ds(sid * 128, 128)])

    @pl.when(sid == 7)
    def _():
        pltpu.sync_copy(rt_hbm.at[pl.ds(7 * 128, N_REL_ - 7 * 128)],
                        tab_sh.at[pl.ds(7 * 128, N_REL_ - 7 * 128)])

    zf = jnp.zeros((16,), jnp.float32)

    @pl.loop(0, CH)
    def _(i):
        for c8 in range(8):
            rows0[i, pl.ds(c8 * 16, 16)] = zf

    for q in range(ACC_SLICE // CH):
        pltpu.sync_copy(rows0,
                        acc_sh.at[pl.ds(sid * ACC_SLICE + q * CH, CH)])

    zi = jnp.zeros((16,), jnp.int32)

    @pl.loop(0, NCH)
    def _(i):
        for c8 in range(8):
            seg_v[i, pl.ds(c8 * 16, 16)] = zi

    ones = jnp.full((16,), 1, jnp.int32)
    OWIN = 512

    def win_body(w, bcnt0):
        pltpu.sync_copy(off_hbm.at[pl.ds(w * OWIN, OWIN)], off_win)

        def mk_body(k, bcnt):
            offv = off_win[pl.ds(k * 16, 16)]
            inside = jnp.logical_and(offv > e_lo, offv < e_lo + EW)
            p = offv - e_lo
            prow = p // CH
            pcol = p - prow * CH
            plsc.addupdate_scatter(seg_v, [prow, pcol], ones, mask=inside)
            return bcnt + jnp.where(offv <= e_lo, 1, 0)

        return pl.loop(0, OWIN // 16, init_carry=bcnt0)(mk_body)

    bcnt = pl.loop(0, N_SEL_ // OWIN,
                   init_carry=jnp.zeros((16,), jnp.int32))(win_body)
    base = jnp.sum(bcnt) - 1

    def cs_body(i, carry):
        for c8 in range(8):
            v = seg_v[i, pl.ds(c8 * 16, 16)]
            c = plsc.cumsum(v) + carry
            seg_v[i, pl.ds(c8 * 16, 16)] = c + base
            carry = c[15]
        return carry

    pl.loop(0, NCH, init_carry=jnp.int32(0))(cs_body)

    for h in range(SEL_W // CH):
        b = wid * SEL_W + h * CH
        pltpu.sync_copy(sel_hbm.at[pl.ds(b, CH)], idx1_v)
        pltpu.async_copy(et_hbm.at[idx1_v], rows1, sem_g[1]).wait()
        pltpu.sync_copy(rows1, rows_out_hbm.at[pl.ds(b, CH)])

    plsc.subcore_barrier()

    def pipe_pass(src_ref, idx2_hbm):
        pltpu.sync_copy(idx2_hbm.at[wid * NCH], ei_b[0])
        pltpu.async_copy(src_ref.at[ei_b[0]], rows_b[0], sem_g[0])
        pltpu.async_copy(idx2_hbm.at[wid * NCH + 1], ei_b[1], sem_i[1])

        @pl.loop(0, NCH // NBUF)
        def _(q):
            for b in range(NBUF):
                ci = q * NBUF + b
                bj = 1 - b
                drain(sem_g[b], rows_b[b])
                pltpu.async_copy(rows_b[b], acc_sh.at[seg_v.at[ci]],
                                 sem_s[b], add=True)
                cj = ci + 1

                @pl.when(cj < NCH)
                def _():
                    @pl.when(ci >= 1)
                    def _():
                        drain(sem_s[bj], rows_b[bj])
                    pltpu.make_async_copy(
                        idx2_hbm.at[wid * NCH], ei_b[bj], sem_i[bj]).wait()
                    pltpu.async_copy(src_ref.at[ei_b[bj]],
                                     rows_b[bj], sem_g[bj])

                    @pl.when(cj + 1 < NCH)
                    def _():
                        pltpu.async_copy(idx2_hbm.at[wid * NCH + cj + 1],
                                         ei_b[b], sem_i[b])

        for b in range(NBUF):
            drain(sem_s[b], rows_b[b])

    pipe_pass(et_hbm, ne_hbm)

    plsc.subcore_barrier()
    o = cid * N_SEL_ + sid * ACC_SLICE
    pltpu.sync_copy(acc_sh.at[pl.ds(sid * ACC_SLICE, ACC_SLICE)],
                    pe_hbm.at[pl.ds(o, ACC_SLICE)])

    @pl.loop(0, CH)
    def _(i):
        for c8 in range(8):
            rows0[i, pl.ds(c8 * 16, 16)] = zf

    for q in range(ACC_SLICE // CH):
        pltpu.sync_copy(rows0,
                        acc_sh.at[pl.ds(sid * ACC_SLICE + q * CH, CH)])
    plsc.subcore_barrier()

    pipe_pass(tab_sh, nr_hbm)

    plsc.subcore_barrier()
    pltpu.sync_copy(acc_sh.at[pl.ds(sid * ACC_SLICE, ACC_SLICE)],
                    pr_hbm.at[pl.ds(o, ACC_SLICE)])


def _sc_bags(neighbor_entities, neighbor_relations, offsets, entities,
             entity_table, relation_table):
    f32 = jnp.float32
    mesh = plsc.VectorSubcoreMesh(core_axis_name="c", subcore_axis_name="s")
    out_type = (jax.ShapeDtypeStruct((2 * N_SEL_, D_), f32),
                jax.ShapeDtypeStruct((2 * N_SEL_, D_), f32),
                jax.ShapeDtypeStruct((N_SEL_, D_), f32))
    scratch = [
        pltpu.VMEM((512,), jnp.int32),
        pltpu.VMEM((NCH, CH), jnp.int32),
        pltpu.VMEM((CH,), jnp.int32),
        pltpu.VMEM((CH,), jnp.int32),
        pltpu.VMEM((CH,), jnp.int32),
        pltpu.VMEM((CH, D_), f32),
        pltpu.VMEM((CH, D_), f32),
        pltpu.VMEM_SHARED((N_SEL_, D_), f32),
        pltpu.VMEM_SHARED((N_REL_, D_), f32),
        pltpu.SemaphoreType.DMA,
        pltpu.SemaphoreType.DMA,
        pltpu.SemaphoreType.DMA,
        pltpu.SemaphoreType.DMA,
        pltpu.SemaphoreType.DMA,
        pltpu.SemaphoreType.DMA,
    ]
    ne2 = neighbor_entities.reshape(NE_ // CH, CH)
    nr2 = neighbor_relations.reshape(NE_ // CH, CH)
    return pl.kernel(
        _sc_body, out_type=out_type, mesh=mesh, scratch_types=scratch,
        compiler_params=pltpu.CompilerParams(needs_layout_passes=False))(
        ne2, nr2, offsets, entities, entity_table, relation_table)


def _tc_node_body(r_ref, pe0_ref, pe1_ref, pr0_ref, pr1_ref,
                  off0_ref, off1_ref, wtT_ref, wnaT_ref, wnbT_ref,
                  bsum_ref, out_ref):
    cnt = (off1_ref[...] - off0_ref[...]).astype(jnp.float32)
    inv = 1.0 / jnp.maximum(cnt, 1.0)
    mean_e = (pe0_ref[...] + pe1_ref[...]) * inv
    mean_r = (pr0_ref[...] + pr1_ref[...]) * inv
    acc = jnp.dot(r_ref[...], wtT_ref[...],
                  preferred_element_type=jnp.float32)
    acc += jnp.dot(mean_e, wnaT_ref[...], preferred_element_type=jnp.float32)
    acc += jnp.dot(mean_r, wnbT_ref[...], preferred_element_type=jnp.float32)
    out_ref[...] = jnp.maximum(acc + bsum_ref[...], 0.0)


def _tc_pair_body(node_ref, wrT_ref, wpT_ref, br_ref, bpp_ref, out_ref):
    pair = jnp.dot(node_ref[...], wrT_ref[...],
                   preferred_element_type=jnp.float32) + br_ref[...]
    pair = jnp.maximum(pair, 0.0)
    out_ref[...] = jnp.dot(pair, wpT_ref[...],
                           preferred_element_type=jnp.float32) + bpp_ref[...]


def kernel(entities, neighbor_entities, neighbor_relations, offsets,
           entity_table, relation_table, Wn, bn, Wt, bt, Wr, br, Wp, bp):
    f32 = jnp.float32

    part_e, part_r, rows = _sc_bags(neighbor_entities, neighbor_relations,
                                    offsets, entities,
                                    entity_table, relation_table)

    offsets_ext = jnp.concatenate([offsets, jnp.full((8,), NE_, jnp.int32)])
    off0 = offsets_ext[:N_SEL_, None]
    off1 = offsets_ext[1:N_SEL_ + 1, None]
    wtT = Wt.T
    wnaT = Wn[:, :D_].T
    wnbT = Wn[:, D_:].T
    bsum = (bt + bn)[None, :]

    node = pl.pallas_call(
        _tc_node_body,
        out_shape=jax.ShapeDtypeStruct((N_SEL_, H_), f32),
    )(rows, part_e[:N_SEL_], part_e[N_SEL_:], part_r[:N_SEL_],
      part_r[N_SEL_:], off0, off1, wtT, wnaT, wnbT, bsum)

    node2 = node.reshape(N_SEL_ // 2, 2 * H_)
    wrT = Wr.T
    wpT = jnp.pad(Wp, ((0, 1), (0, 0))).T
    bpp = jnp.pad(bp, (0, 1))[None, :]

    scores = pl.pallas_call(
        _tc_pair_body,
        out_shape=jax.ShapeDtypeStruct((N_SEL_ // 2, N_REL_), f32),
    )(node2, wrT, wpT, br[None, :], bpp)

    return scores[:, :N_REL_ - 1]
